# baseline (speedup 1.0000x reference)
import functools

import jax
import jax.numpy as jnp
from jax import lax
from jax.experimental import pallas as pl
from jax.experimental.pallas import tpu as pltpu
from jax.experimental.pallas import tpu_sc as plsc

B, S = 16, 1500
CODES, SW, W = 1024, 768, 1024
NT = B * S
C = 48
NCHUNKS = NT // C
NWORKERS = 32
LANES = 16


def _mm_body(a_ref, b_ref, bias_ref, o_ref):
    o_ref[...] = (
        jnp.dot(a_ref[...], b_ref[...], preferred_element_type=jnp.float32,
                precision=lax.Precision.HIGHEST)
        + bias_ref[...]
    )


def _project_table(main_w, e2h_w, e2h_b):
    return pl.pallas_call(
        _mm_body,
        out_shape=jax.ShapeDtypeStruct((CODES, W), jnp.float32),
    )(main_w, e2h_w, e2h_b.reshape(1, W))


def _sc_body(table, idxf, pidxf, pos_emb, out,
             idx_v, pidx_v, rows_v, pos_v, sem_t, sem_p):
    wid = lax.axis_index("s") * 2 + lax.axis_index("c")
    nch = (NCHUNKS - wid + NWORKERS - 1) // NWORKERS

    def chunk(k, carry):
        base = (wid + NWORKERS * k) * C
        pltpu.sync_copy(idxf.at[pl.ds(base, C)], idx_v)
        pltpu.sync_copy(pidxf.at[pl.ds(base, C)], pidx_v)
        cp_t = pltpu.async_copy(table.at[idx_v], rows_v, sem_t)
        cp_p = pltpu.async_copy(pos_emb.at[pidx_v], pos_v, sem_p)
        cp_t.wait()
        cp_p.wait()

        def row_add(i, c2):
            for j in range(W // LANES):
                plsc.addupdate(rows_v.at[i, pl.ds(LANES * j, LANES)],
                               pos_v[i, pl.ds(LANES * j, LANES)])
            return c2

        lax.fori_loop(0, C, row_add, 0, unroll=False)
        pltpu.sync_copy(rows_v, out.at[pl.ds(base, C)])
        return carry

    lax.fori_loop(0, nch, chunk, 0, unroll=False)


@functools.partial(
    pl.kernel,
    out_type=jax.ShapeDtypeStruct((NT, W), jnp.float32),
    mesh=plsc.VectorSubcoreMesh(core_axis_name="c", subcore_axis_name="s"),
    scratch_types=[
        pltpu.VMEM((C,), jnp.int32),
        pltpu.VMEM((C,), jnp.int32),
        pltpu.VMEM((C, W), jnp.float32),
        pltpu.VMEM((C, W), jnp.float32),
        pltpu.SemaphoreType.DMA,
        pltpu.SemaphoreType.DMA,
    ],
)
def _sc_gather_add(table, idxf, pidxf, pos_emb, out,
                   idx_v, pidx_v, rows_v, pos_v, sem_t, sem_p):
    _sc_body(table, idxf, pidxf, pos_emb, out,
             idx_v, pidx_v, rows_v, pos_v, sem_t, sem_p)


def kernel(Stoks, xenc, main_w, special_w, e2h_w, e2h_b, pos_emb):
    proj = _project_table(main_w, e2h_w, e2h_b)
    table = jnp.concatenate([proj, special_w], axis=0)
    idxf = Stoks.reshape(NT).astype(jnp.int32)
    pidxf = jnp.arange(NT, dtype=jnp.int32) % S
    out = _sc_gather_add(table, idxf, pidxf, pos_emb)
    return (out.reshape(B, S, W).astype(xenc.dtype), 0)
